# baseline (speedup 1.0000x reference)
import functools

import jax
import jax.numpy as jnp
from jax import lax
from jax.experimental import pallas as pl
from jax.experimental.pallas import tpu as pltpu
from jax.experimental.pallas import tpu_sc as plsc

R = 128
C = 32768
NC = 2
NS = 16
L = 16
RPW = R // (NC * NS)
RPC = R // NC
K = 8
STEPS = C // L
GROUPS = STEPS // K

_BIG = 2**30


def _shuffle(x, d):
    perm = lax.iota(jnp.int32, L) ^ d
    return x.at[perm].get(mode="promise_in_bounds")


def _row_argmax(buf, slot):
    iota = lax.iota(jnp.int32, L)
    neg_inf = jnp.full((L,), -jnp.inf, jnp.float32)
    ms = [neg_inf for _ in range(K)]
    ivs = [iota + k * L for k in range(K)]
    bis = list(ivs)

    def body(g, carry):
        ms, bis, ivs = carry
        base = g * (K * L)
        new_ms, new_bis, new_ivs = [], [], []
        for k in range(K):
            v = buf[slot, pl.ds(base + k * L, L)]
            p = v > ms[k]
            new_ms.append(jnp.where(p, v, ms[k]))
            new_bis.append(jnp.where(p, ivs[k], bis[k]))
            new_ivs.append(ivs[k] + K * L)
        return tuple(new_ms), tuple(new_bis), tuple(new_ivs)

    ms, bis, _ = lax.fori_loop(0, GROUPS, body, (tuple(ms), tuple(bis), tuple(ivs)))

    m = ms[0]
    for k in range(1, K):
        m = jnp.maximum(m, ms[k])
    for d in (1, 2, 4, 8):
        m = jnp.maximum(m, _shuffle(m, d))
    cand = jnp.where(ms[0] == m, bis[0], _BIG)
    for k in range(1, K):
        cand = jnp.minimum(cand, jnp.where(ms[k] == m, bis[k], _BIG))
    for d in (1, 2, 4, 8):
        cand = jnp.minimum(cand, _shuffle(cand, d))
    return cand


def _body(x_hbm, out_hbm, buf, res_v, sem0, sem1):
    c = lax.axis_index("c")
    s = lax.axis_index("s")
    wid = c * NS + s
    base_row = wid * RPW
    sems = (sem0, sem1)

    copies = [pltpu.async_copy(x_hbm.at[base_row], buf.at[0], sems[0])]
    iota = lax.iota(jnp.int32, L)
    res = jnp.full((L,), -1.0, jnp.float32)
    for r in range(RPW):
        if r + 1 < RPW:
            copies.append(
                pltpu.async_copy(
                    x_hbm.at[base_row + r + 1], buf.at[(r + 1) % 2], sems[(r + 1) % 2]
                )
            )
        copies[r].wait()
        idxv = _row_argmax(buf, r % 2)
        res = jnp.where(iota == r, idxv.astype(jnp.float32), res)

    res_v[...] = res
    pltpu.sync_copy(res_v, out_hbm.at[wid])


_sc_argmax = pl.kernel(
    _body,
    out_type=jax.ShapeDtypeStruct((NC * NS, L), jnp.float32),
    mesh=plsc.VectorSubcoreMesh(core_axis_name="c", subcore_axis_name="s"),
    scratch_types=[
        pltpu.VMEM((2, C), jnp.float32),
        pltpu.VMEM((L,), jnp.float32),
        pltpu.SemaphoreType.DMA,
        pltpu.SemaphoreType.DMA,
    ],
)


def kernel(x):
    return _sc_argmax(x)[:, :RPW].reshape(R, 1)


# speedup vs baseline: 1.0383x; 1.0383x over previous
import jax
import jax.numpy as jnp
from jax import lax
from jax.experimental import pallas as pl
from jax.experimental.pallas import tpu as pltpu
from jax.experimental.pallas import tpu_sc as plsc

R = 128
C = 32768
NC = 2
NS = 16
L = 16
RPW = R // (NC * NS)
K = 8
STEPS = C // L
GROUPS = STEPS // K

_BIG = 2**30


def _shuffle(x, d):
    perm = lax.iota(jnp.int32, L) ^ d
    return x.at[perm].get(mode="promise_in_bounds")


def _row_argmax(buf, slot):
    iota = lax.iota(jnp.int32, L)
    neg_inf = jnp.full((L,), -jnp.inf, jnp.float32)
    zero = jnp.zeros((L,), jnp.int32)
    ms = tuple(neg_inf for _ in range(K))
    gs = tuple(zero for _ in range(K))

    def body(g, carry):
        ms, gs, gvec = carry
        base = g * (K * L)
        new_ms, new_gs = [], []
        for k in range(K):
            v = buf[slot, pl.ds(base + k * L, L)]
            p = v > ms[k]
            new_ms.append(jnp.maximum(ms[k], v))
            new_gs.append(jnp.where(p, gvec, gs[k]))
        return tuple(new_ms), tuple(new_gs), gvec + 1

    ms, gs, _ = lax.fori_loop(0, GROUPS, body, (ms, gs, zero))

    m = ms[0]
    for k in range(1, K):
        m = jnp.maximum(m, ms[k])
    for d in (1, 2, 4, 8):
        m = jnp.maximum(m, _shuffle(m, d))
    cand = jnp.full((L,), _BIG, jnp.int32)
    for k in range(K):
        idx_k = lax.bitwise_or(lax.shift_left(gs[k], 7), iota + k * L)
        cand = jnp.minimum(cand, jnp.where(ms[k] == m, idx_k, _BIG))
    for d in (1, 2, 4, 8):
        cand = jnp.minimum(cand, _shuffle(cand, d))
    return cand


def _body(x_hbm, out0, out1, buf, res_v, sem0, sem1):
    c = lax.axis_index("c")
    s = lax.axis_index("s")
    base_row = (c * NS + s) * RPW
    sems = (sem0, sem1)

    copies = [pltpu.async_copy(x_hbm.at[base_row], buf.at[0], sems[0])]
    iota = lax.iota(jnp.int32, L)
    res = jnp.full((L,), -1.0, jnp.float32)
    for r in range(RPW):
        if r + 1 < RPW:
            copies.append(
                pltpu.async_copy(
                    x_hbm.at[base_row + r + 1], buf.at[(r + 1) % 2], sems[(r + 1) % 2]
                )
            )
        copies[r].wait()
        idxv = _row_argmax(buf, r % 2)
        res = jnp.where(iota == r, idxv.astype(jnp.float32), res)

    res_v[...] = res

    @pl.when(c == 0)
    def _():
        pltpu.sync_copy(res_v, out0.at[s])

    @pl.when(c == 1)
    def _():
        pltpu.sync_copy(res_v, out1.at[s])


_sc_argmax = pl.kernel(
    _body,
    out_type=(
        jax.ShapeDtypeStruct((NS, L), jnp.float32),
        jax.ShapeDtypeStruct((NS, L), jnp.float32),
    ),
    mesh=plsc.VectorSubcoreMesh(core_axis_name="c", subcore_axis_name="s"),
    scratch_types=[
        pltpu.VMEM((2, C), jnp.float32),
        pltpu.VMEM((L,), jnp.float32),
        pltpu.SemaphoreType.DMA,
        pltpu.SemaphoreType.DMA,
    ],
)


def kernel(x):
    y0, y1 = _sc_argmax(x)
    y = jnp.concatenate([y0[:, :RPW], y1[:, :RPW]], axis=0)
    return y.reshape(R, 1)


# speedup vs baseline: 1.0790x; 1.0392x over previous
import jax
import jax.numpy as jnp
from jax import lax
from jax.experimental import pallas as pl
from jax.experimental.pallas import tpu as pltpu
from jax.experimental.pallas import tpu_sc as plsc

R = 128
C = 32768
NC = 2
NS = 16
L = 16
RPW = R // (NC * NS)
K = 8
STEPS = C // L
GROUPS = STEPS // K

_BIG = 2**30


def _shuffle(x, d):
    perm = lax.iota(jnp.int32, L) ^ d
    return x.at[perm].get(mode="promise_in_bounds")


def _row_argmax(buf, off):
    iota = lax.iota(jnp.int32, L)
    neg_inf = jnp.full((L,), -jnp.inf, jnp.float32)
    zero = jnp.zeros((L,), jnp.int32)
    ms = tuple(neg_inf for _ in range(K))
    gs = tuple(zero for _ in range(K))

    def body(g, carry):
        ms, gs, gvec = carry
        base = off + g * (K * L)
        new_ms, new_gs = [], []
        for k in range(K):
            v = buf[pl.ds(base + k * L, L)]
            p = v > ms[k]
            new_ms.append(jnp.maximum(ms[k], v))
            new_gs.append(jnp.where(p, gvec, gs[k]))
        return tuple(new_ms), tuple(new_gs), gvec + 1

    ms, gs, _ = lax.fori_loop(0, GROUPS, body, (ms, gs, zero))

    m = ms[0]
    for k in range(1, K):
        m = jnp.maximum(m, ms[k])
    for d in (1, 2, 4, 8):
        m = jnp.maximum(m, _shuffle(m, d))
    cand = jnp.full((L,), _BIG, jnp.int32)
    for k in range(K):
        idx_k = lax.bitwise_or(lax.shift_left(gs[k], 7), iota + k * L)
        cand = jnp.minimum(cand, jnp.where(ms[k] == m, idx_k, _BIG))
    for d in (1, 2, 4, 8):
        cand = jnp.minimum(cand, _shuffle(cand, d))
    return cand


def _body(x_hbm, out0, out1, buf, res_v, sem0, sem1):
    c = lax.axis_index("c")
    s = lax.axis_index("s")
    base_row = (c * NS + s) * RPW
    sems = (sem0, sem1)

    pltpu.async_copy(x_hbm.at[base_row], buf.at[pl.ds(0, C)], sems[0])
    iota = lax.iota(jnp.int32, L)

    def row_body(r, res):
        even = lax.rem(r, 2) == 0

        @pl.when(r < RPW - 1)
        def _():
            nxt = base_row + r + 1

            @pl.when(even)
            def _():
                pltpu.async_copy(x_hbm.at[nxt], buf.at[pl.ds(C, C)], sems[1])

            @pl.when(jnp.logical_not(even))
            def _():
                pltpu.async_copy(x_hbm.at[nxt], buf.at[pl.ds(0, C)], sems[0])

        @pl.when(even)
        def _():
            pltpu.make_async_copy(x_hbm.at[base_row], buf.at[pl.ds(0, C)], sems[0]).wait()

        @pl.when(jnp.logical_not(even))
        def _():
            pltpu.make_async_copy(x_hbm.at[base_row], buf.at[pl.ds(C, C)], sems[1]).wait()

        off = jnp.where(even, 0, C)
        idxv = _row_argmax(buf, off)
        return jnp.where(iota == r, idxv.astype(jnp.float32), res)

    res = lax.fori_loop(0, RPW, row_body, jnp.full((L,), -1.0, jnp.float32))
    res_v[...] = res

    @pl.when(c == 0)
    def _():
        pltpu.sync_copy(res_v, out0.at[s])

    @pl.when(c == 1)
    def _():
        pltpu.sync_copy(res_v, out1.at[s])


_sc_argmax = pl.kernel(
    _body,
    out_type=(
        jax.ShapeDtypeStruct((NS, L), jnp.float32),
        jax.ShapeDtypeStruct((NS, L), jnp.float32),
    ),
    mesh=plsc.VectorSubcoreMesh(core_axis_name="c", subcore_axis_name="s"),
    scratch_types=[
        pltpu.VMEM((2 * C,), jnp.float32),
        pltpu.VMEM((L,), jnp.float32),
        pltpu.SemaphoreType.DMA,
        pltpu.SemaphoreType.DMA,
    ],
)


def kernel(x):
    y0, y1 = _sc_argmax(x)
    y = jnp.concatenate([y0[:, :RPW], y1[:, :RPW]], axis=0)
    return y.reshape(R, 1)
